# baseline (speedup 1.0000x reference)
import jax
import jax.numpy as jnp
from jax import lax
from jax.experimental import pallas as pl
from jax.experimental.pallas import tpu as pltpu
from jax.experimental.pallas import tpu_sc as plsc

_B, _T = 16, 576
_ROWS = _B * _T
_K = 1024
_D = 64
_TILE = 512
_KC = 128
_NC, _NS = 2, 16
_NW = _NC * _NS
_BPW = _ROWS // _NW
_CHUNK = 96
_NCHUNK = _BPW // _CHUNK


def _argmin_body(x_ref, em2_ref, e2_ref, idx_ref):
    x = x_ref[...]
    xsq = x * x
    rowsq = lax.dot_general(jnp.ones((1, _D), jnp.float32), xsq,
                            (((1,), (1,)), ((), ())),
                            preferred_element_type=jnp.float32)
    rowf = lax.broadcasted_iota(jnp.int32, (_KC, _TILE), 0).astype(jnp.float32)
    run_min = None
    for c in range(_K // _KC):
        sl = pl.ds(c * _KC, _KC)
        s = lax.dot_general(em2_ref[sl, :], x, (((1,), (1,)), ((), ())),
                            preferred_element_type=jnp.float32)
        d2 = (rowsq + s) + e2_ref[sl, :]
        cmin = jnp.min(d2, axis=0, keepdims=True)
        row = rowf + float(c * _KC)
        cidx = jnp.min(jnp.where(d2 == cmin, row, float(_K)), axis=0,
                       keepdims=True)
        if run_min is None:
            run_min, run_idx = cmin, cidx
        else:
            better = cmin < run_min
            run_min = jnp.where(better, cmin, run_min)
            run_idx = jnp.where(better, cidx, run_idx)
    idx_ref[...] = run_idx.reshape(_TILE).astype(jnp.int32)


def _nearest_indices(flat, em2, e2):
    return pl.pallas_call(
        _argmin_body,
        grid=(_ROWS // _TILE,),
        in_specs=[
            pl.BlockSpec((_TILE, _D), lambda i: (i, 0)),
            pl.BlockSpec((_K, _D), lambda i: (0, 0)),
            pl.BlockSpec((_K, 1), lambda i: (0, 0)),
        ],
        out_specs=pl.BlockSpec((_TILE,), lambda i: (i,)),
        out_shape=jax.ShapeDtypeStruct((_ROWS,), jnp.int32),
    )(flat, em2, e2)


def _gather_body(table_hbm, idx_hbm, out_hbm, idx_v, rows_v, sem):
    wid = lax.axis_index("s") * _NC + lax.axis_index("c")
    base = wid * _BPW
    pltpu.sync_copy(idx_hbm.at[pl.ds(base, _BPW)], idx_v)
    copies = [
        pltpu.async_copy(
            table_hbm.at[idx_v.at[pl.ds(c * _CHUNK, _CHUNK)]],
            rows_v.at[pl.ds(c * _CHUNK, _CHUNK)],
            sem,
        )
        for c in range(_NCHUNK)
    ]
    for cp in copies:
        cp.wait()
    b = wid // 2
    h = wid % 2
    pltpu.sync_copy(rows_v, out_hbm.at[b, pl.ds(h * _BPW, _BPW)])


def _gather_rows(embedding, idx):
    return pl.kernel(
        _gather_body,
        out_type=jax.ShapeDtypeStruct((_B, _T, _D), jnp.float32),
        mesh=plsc.VectorSubcoreMesh(core_axis_name="c", subcore_axis_name="s"),
        compiler_params=pltpu.CompilerParams(use_tc_tiling_on_sc=False),
        scratch_types=[
            pltpu.VMEM((_BPW,), jnp.int32),
            pltpu.VMEM((_BPW, _D), jnp.float32),
            pltpu.SemaphoreType.DMA,
        ],
    )(embedding, idx)


def kernel(encoded, embedding):
    bsz, T, dims = encoded.shape
    flat = encoded.reshape(bsz * T, dims)
    em2 = embedding * -2.0
    e2 = jnp.sum(embedding * embedding, axis=1)[:, None]
    idx = _nearest_indices(flat, em2, e2)
    return _gather_rows(embedding, idx)


# speedup vs baseline: 1.0435x; 1.0435x over previous
import jax
import jax.numpy as jnp
from jax import lax
from jax.experimental import pallas as pl
from jax.experimental.pallas import tpu as pltpu
from jax.experimental.pallas import tpu_sc as plsc

_B, _T = 16, 576
_ROWS = _B * _T
_K = 1024
_D = 64
_TILE = 512
_KC = 128
_NC, _NS = 2, 16
_NW = _NC * _NS
_BPW = _ROWS // _NW
_CHUNK = 96
_NCHUNK = _BPW // _CHUNK


def _argmin_body(x_ref, em2_ref, idx_ref):
    x = x_ref[...]
    xsq = x * x
    rowsq = lax.dot_general(jnp.ones((1, _D), jnp.float32), xsq,
                            (((1,), (1,)), ((), ())),
                            preferred_element_type=jnp.float32)
    rowf = lax.broadcasted_iota(jnp.int32, (_KC, _TILE), 0).astype(jnp.float32)
    run_min = None
    for c in range(_K // _KC):
        sl = pl.ds(c * _KC, _KC)
        em2c = em2_ref[sl, :]
        s = lax.dot_general(em2c, x, (((1,), (1,)), ((), ())),
                            preferred_element_type=jnp.float32)
        e2c = lax.dot_general(em2c * em2c, jnp.ones((1, _D), jnp.float32),
                              (((1,), (1,)), ((), ())),
                              preferred_element_type=jnp.float32) * 0.25
        d2 = (rowsq + s) + e2c
        cmin = jnp.min(d2, axis=0, keepdims=True)
        row = rowf + float(c * _KC)
        cidx = jnp.min(jnp.where(d2 == cmin, row, float(_K)), axis=0,
                       keepdims=True)
        if run_min is None:
            run_min, run_idx = cmin, cidx
        else:
            better = cmin < run_min
            run_min = jnp.where(better, cmin, run_min)
            run_idx = jnp.where(better, cidx, run_idx)
    idx_ref[...] = run_idx.reshape(_TILE).astype(jnp.int32)


def _nearest_indices(flat, em2):
    return pl.pallas_call(
        _argmin_body,
        grid=(_ROWS // _TILE,),
        in_specs=[
            pl.BlockSpec((_TILE, _D), lambda i: (i, 0)),
            pl.BlockSpec((_K, _D), lambda i: (0, 0)),
        ],
        out_specs=pl.BlockSpec((_TILE,), lambda i: (i,)),
        out_shape=jax.ShapeDtypeStruct((_ROWS,), jnp.int32),
    )(flat, em2)


def _gather_body(table_hbm, idx_hbm, out_hbm, idx_v, rows_v, sem):
    wid = lax.axis_index("s") * _NC + lax.axis_index("c")
    base = wid * _BPW
    pltpu.sync_copy(idx_hbm.at[pl.ds(base, _BPW)], idx_v)
    copies = [
        pltpu.async_copy(
            table_hbm.at[idx_v.at[pl.ds(c * _CHUNK, _CHUNK)]],
            rows_v.at[pl.ds(c * _CHUNK, _CHUNK)],
            sem,
        )
        for c in range(_NCHUNK)
    ]
    for cp in copies:
        cp.wait()
    b = wid // 2
    h = wid % 2
    pltpu.sync_copy(rows_v, out_hbm.at[b, pl.ds(h * _BPW, _BPW)])


def _gather_rows(embedding, idx):
    return pl.kernel(
        _gather_body,
        out_type=jax.ShapeDtypeStruct((_B, _T, _D), jnp.float32),
        mesh=plsc.VectorSubcoreMesh(core_axis_name="c", subcore_axis_name="s"),
        compiler_params=pltpu.CompilerParams(use_tc_tiling_on_sc=False),
        scratch_types=[
            pltpu.VMEM((_BPW,), jnp.int32),
            pltpu.VMEM((_BPW, _D), jnp.float32),
            pltpu.SemaphoreType.DMA,
        ],
    )(embedding, idx)


def kernel(encoded, embedding):
    bsz, T, dims = encoded.shape
    flat = encoded.reshape(bsz * T, dims)
    em2 = embedding * -2.0
    idx = _nearest_indices(flat, em2)
    return _gather_rows(embedding, idx)


# speedup vs baseline: 1.0890x; 1.0436x over previous
import jax
import jax.numpy as jnp
from jax import lax
from jax.experimental import pallas as pl
from jax.experimental.pallas import tpu as pltpu
from jax.experimental.pallas import tpu_sc as plsc

_B, _T = 16, 576
_ROWS = _B * _T
_K = 1024
_D = 64
_BT = 2
_TILE = _BT * _T
_KC = 128
_NC, _NS = 2, 16
_NW = _NC * _NS
_BPW = _ROWS // _NW
_CHUNK = 96
_NCHUNK = _BPW // _CHUNK


def _argmin_group(x, em2_ref, rowf):
    xsq = x * x
    rowsq = lax.dot_general(jnp.ones((1, _D), jnp.float32), xsq,
                            (((1,), (1,)), ((), ())),
                            preferred_element_type=jnp.float32)
    run_min = None
    for c in range(_K // _KC):
        sl = pl.ds(c * _KC, _KC)
        em2c = em2_ref[sl, :]
        s = lax.dot_general(em2c, x, (((1,), (1,)), ((), ())),
                            preferred_element_type=jnp.float32)
        e2c = lax.dot_general(em2c * em2c, jnp.ones((1, _D), jnp.float32),
                              (((1,), (1,)), ((), ())),
                              preferred_element_type=jnp.float32) * 0.25
        d2 = (rowsq + s) + e2c
        cmin = jnp.min(d2, axis=0, keepdims=True)
        row = rowf + float(c * _KC)
        cidx = jnp.min(jnp.where(d2 == cmin, row, float(_K)), axis=0,
                       keepdims=True)
        if run_min is None:
            run_min, run_idx = cmin, cidx
        else:
            better = cmin < run_min
            run_min = jnp.where(better, cmin, run_min)
            run_idx = jnp.where(better, cidx, run_idx)
    return run_idx


def _argmin_body(x_ref, em2_ref, idx_ref):
    rowf = lax.broadcasted_iota(jnp.int32, (_KC, _TILE), 0).astype(jnp.float32)

    def group(g, _):
        x = x_ref[pl.ds(g * _BT, _BT)].reshape(_TILE, _D)
        run_idx = _argmin_group(x, em2_ref, rowf)
        off = pl.multiple_of(g * _TILE, _TILE)
        idx_ref[pl.ds(off, _TILE)] = run_idx.reshape(_TILE).astype(jnp.int32)
        return 0

    lax.fori_loop(0, _B // _BT, group, 0)


def _nearest_indices(encoded, em2):
    return pl.pallas_call(
        _argmin_body,
        in_specs=[
            pl.BlockSpec((_B, _T, _D), lambda: (0, 0, 0)),
            pl.BlockSpec((_K, _D), lambda: (0, 0)),
        ],
        out_specs=pl.BlockSpec((_ROWS,), lambda: (0,)),
        out_shape=jax.ShapeDtypeStruct((_ROWS,), jnp.int32),
    )(encoded, em2)


def _gather_body(table_hbm, idx_hbm, out_hbm, idx_v, rows_v, sem):
    wid = lax.axis_index("s") * _NC + lax.axis_index("c")
    base = wid * _BPW
    pltpu.sync_copy(idx_hbm.at[pl.ds(base, _BPW)], idx_v)
    copies = [
        pltpu.async_copy(
            table_hbm.at[idx_v.at[pl.ds(c * _CHUNK, _CHUNK)]],
            rows_v.at[pl.ds(c * _CHUNK, _CHUNK)],
            sem,
        )
        for c in range(_NCHUNK)
    ]
    for cp in copies:
        cp.wait()
    b = wid // 2
    h = wid % 2
    pltpu.sync_copy(rows_v, out_hbm.at[b, pl.ds(h * _BPW, _BPW)])


def _gather_rows(embedding, idx):
    return pl.kernel(
        _gather_body,
        out_type=jax.ShapeDtypeStruct((_B, _T, _D), jnp.float32),
        mesh=plsc.VectorSubcoreMesh(core_axis_name="c", subcore_axis_name="s"),
        compiler_params=pltpu.CompilerParams(use_tc_tiling_on_sc=False),
        scratch_types=[
            pltpu.VMEM((_BPW,), jnp.int32),
            pltpu.VMEM((_BPW, _D), jnp.float32),
            pltpu.SemaphoreType.DMA,
        ],
    )(embedding, idx)


def kernel(encoded, embedding):
    em2 = embedding * -2.0
    idx = _nearest_indices(encoded, em2)
    return _gather_rows(embedding, idx)
